# baseline (speedup 1.0000x reference)
import jax
import jax.numpy as jnp
from jax.experimental import pallas as pl


def _l2n(a, axis, eps=1e-12):
    norm = jnp.linalg.norm(a, ord=2, axis=axis, keepdims=True)
    return a / jnp.maximum(norm, eps)


def kernel(x, memory):
    mem_norm = _l2n(memory, axis=1)
    x_norm = _l2n(x, axis=1)
    sim = jnp.matmul(x_norm.astype(jnp.bfloat16), mem_norm.T.astype(jnp.bfloat16),
                     preferred_element_type=jnp.float32)
    topk_sim, topk_idx = jax.lax.top_k(sim, 10)
    weights = jax.nn.softmax(topk_sim, axis=1)
    selected_mem = memory[topk_idx]
    recon = jnp.sum(weights[..., None] * selected_mem, axis=1)
    return recon


# speedup vs baseline: 1.2075x; 1.2075x over previous
import functools

import jax
import jax.numpy as jnp
from jax.experimental import pallas as pl
from jax.experimental.pallas import tpu as pltpu

B_BLK = 512
M_CHUNK = 2048
TOPK = 10
KPAD = 16
NEG = -1e30


def _topk_kernel(x_ref, mem_ref, vals_ref, idx_ref, rv_ref, ri_ref, *,
                 n_chunks, m_valid):
    j = pl.program_id(1)

    @pl.when(j == 0)
    def _init():
        rv_ref[:] = jnp.full((B_BLK, KPAD), NEG, jnp.float32)
        ri_ref[:] = jnp.zeros((B_BLK, KPAD), jnp.int32)

    x = x_ref[:]
    xn = x * jax.lax.rsqrt(
        jnp.maximum(jnp.sum(x * x, axis=1, keepdims=True), 1e-24))
    mem = mem_ref[:]
    mn = mem * jax.lax.rsqrt(
        jnp.maximum(jnp.sum(mem * mem, axis=1, keepdims=True), 1e-24))
    sim = jax.lax.dot_general(
        xn, mn, (((1,), (1,)), ((), ())),
        preferred_element_type=jnp.float32)

    col = jax.lax.broadcasted_iota(jnp.int32, (B_BLK, M_CHUNK), 1)
    gcol = col + j * M_CHUNK
    sim = jnp.where(gcol < m_valid, sim, NEG)

    cand_v = jnp.concatenate([sim, rv_ref[:]], axis=1)
    cand_i = jnp.concatenate([gcol, ri_ref[:]], axis=1)
    W = M_CHUNK + KPAD
    iota = jax.lax.broadcasted_iota(jnp.int32, (B_BLK, W), 1)
    k16 = jax.lax.broadcasted_iota(jnp.int32, (B_BLK, KPAD), 1)

    new_v = jnp.full((B_BLK, KPAD), NEG, jnp.float32)
    new_i = jnp.zeros((B_BLK, KPAD), jnp.int32)
    for k in range(TOPK):
        m = jnp.max(cand_v, axis=1, keepdims=True)
        pos = jnp.min(jnp.where(cand_v == m, iota, W), axis=1, keepdims=True)
        sel = iota == pos
        idx_k = jnp.sum(jnp.where(sel, cand_i, 0), axis=1, keepdims=True)
        new_v = jnp.where(k16 == k, m, new_v)
        new_i = jnp.where(k16 == k, idx_k, new_i)
        cand_v = jnp.where(sel, NEG, cand_v)

    rv_ref[:] = new_v
    ri_ref[:] = new_i

    @pl.when(j == n_chunks - 1)
    def _out():
        vals_ref[:] = new_v
        idx_ref[:] = new_i


def _recon_kernel(vals_ref, idx_ref, mem_ref, out_ref):
    j = pl.program_id(1)

    @pl.when(j == 0)
    def _init():
        out_ref[:] = jnp.zeros_like(out_ref)

    v = vals_ref[:, :TOPK]
    m = jnp.max(v, axis=1, keepdims=True)
    e = jnp.exp(v - m)
    w = e / jnp.sum(e, axis=1, keepdims=True)

    gcol = jax.lax.broadcasted_iota(jnp.int32, (B_BLK, M_CHUNK), 1) \
        + j * M_CHUNK
    oh = jnp.zeros((B_BLK, M_CHUNK), jnp.float32)
    for k in range(TOPK):
        oh = oh + jnp.where(idx_ref[:, k:k + 1] == gcol, w[:, k:k + 1], 0.0)

    out_ref[:] += jax.lax.dot_general(
        oh, mem_ref[:], (((1,), (0,)), ((), ())),
        preferred_element_type=jnp.float32)


@jax.jit
def kernel(x, memory):
    b, d = x.shape
    m, _ = memory.shape
    m_pad = ((m + M_CHUNK - 1) // M_CHUNK) * M_CHUNK
    mem_p = jnp.pad(memory, ((0, m_pad - m), (0, 0)))
    nb = b // B_BLK
    nj = m_pad // M_CHUNK

    vals, idx = pl.pallas_call(
        functools.partial(_topk_kernel, n_chunks=nj, m_valid=m),
        grid=(nb, nj),
        in_specs=[
            pl.BlockSpec((B_BLK, d), lambda i, j: (i, 0)),
            pl.BlockSpec((M_CHUNK, d), lambda i, j: (j, 0)),
        ],
        out_specs=[
            pl.BlockSpec((B_BLK, KPAD), lambda i, j: (i, 0)),
            pl.BlockSpec((B_BLK, KPAD), lambda i, j: (i, 0)),
        ],
        out_shape=[
            jax.ShapeDtypeStruct((b, KPAD), jnp.float32),
            jax.ShapeDtypeStruct((b, KPAD), jnp.int32),
        ],
        scratch_shapes=[
            pltpu.VMEM((B_BLK, KPAD), jnp.float32),
            pltpu.VMEM((B_BLK, KPAD), jnp.int32),
        ],
        compiler_params=pltpu.CompilerParams(
            dimension_semantics=("arbitrary", "arbitrary")),
    )(x, mem_p)

    recon = pl.pallas_call(
        _recon_kernel,
        grid=(nb, nj),
        in_specs=[
            pl.BlockSpec((B_BLK, KPAD), lambda i, j: (i, 0)),
            pl.BlockSpec((B_BLK, KPAD), lambda i, j: (i, 0)),
            pl.BlockSpec((M_CHUNK, d), lambda i, j: (j, 0)),
        ],
        out_specs=pl.BlockSpec((B_BLK, d), lambda i, j: (i, 0)),
        out_shape=jax.ShapeDtypeStruct((b, d), jnp.float32),
        compiler_params=pltpu.CompilerParams(
            dimension_semantics=("arbitrary", "arbitrary")),
    )(vals, idx, mem_p)

    return recon


# speedup vs baseline: 1.2736x; 1.0547x over previous
import functools

import jax
import jax.numpy as jnp
from jax.experimental import pallas as pl
from jax.experimental.pallas import tpu as pltpu

B_BLK = 512
M_CHUNK = 2048
TOPK = 10
KPAD = 16
NEG = -1e30


def _topk_kernel(x_ref, mem_ref, vals_ref, idx_ref, rv_ref, ri_ref, *,
                 n_chunks, m_valid):
    j = pl.program_id(1)

    @pl.when(j == 0)
    def _init():
        rv_ref[:] = jnp.full((B_BLK, KPAD), NEG, jnp.float32)
        ri_ref[:] = jnp.zeros((B_BLK, KPAD), jnp.int32)

    x = x_ref[:]
    xn = x * jax.lax.rsqrt(
        jnp.maximum(jnp.sum(x * x, axis=1, keepdims=True), 1e-24))
    mem = mem_ref[:]
    mn = mem * jax.lax.rsqrt(
        jnp.maximum(jnp.sum(mem * mem, axis=1, keepdims=True), 1e-24))
    sim = jax.lax.dot_general(
        xn, mn, (((1,), (1,)), ((), ())),
        preferred_element_type=jnp.float32)

    col = jax.lax.broadcasted_iota(jnp.int32, (B_BLK, M_CHUNK), 1)
    gcol = col + j * M_CHUNK
    sim = jnp.where(gcol < m_valid, sim, NEG)

    k16 = jax.lax.broadcasted_iota(jnp.int32, (B_BLK, KPAD), 1)
    rv = rv_ref[:]
    ri = ri_ref[:]

    new_v = jnp.full((B_BLK, KPAD), NEG, jnp.float32)
    new_i = jnp.zeros((B_BLK, KPAD), jnp.int32)
    for k in range(TOPK):
        m1 = jnp.max(sim, axis=1, keepdims=True)
        m2 = jnp.max(rv, axis=1, keepdims=True)
        take_sim = m1 >= m2
        pos1 = jnp.min(jnp.where(sim == m1, col, M_CHUNK), axis=1,
                       keepdims=True)
        pos2 = jnp.min(jnp.where(rv == m2, k16, KPAD), axis=1, keepdims=True)
        idx2 = jnp.sum(jnp.where(k16 == pos2, ri, 0), axis=1, keepdims=True)
        idx1 = pos1 + j * M_CHUNK
        new_v = jnp.where(k16 == k, jnp.maximum(m1, m2), new_v)
        new_i = jnp.where(k16 == k, jnp.where(take_sim, idx1, idx2), new_i)
        sim = jnp.where(take_sim & (col == pos1), NEG, sim)
        rv = jnp.where(jnp.logical_not(take_sim) & (k16 == pos2), NEG, rv)

    rv_ref[:] = new_v
    ri_ref[:] = new_i

    @pl.when(j == n_chunks - 1)
    def _out():
        vals_ref[:] = new_v
        idx_ref[:] = new_i


def _recon_kernel(vals_ref, idx_ref, mem_ref, out_ref):
    j = pl.program_id(1)

    @pl.when(j == 0)
    def _init():
        out_ref[:] = jnp.zeros_like(out_ref)

    v = vals_ref[:, :TOPK]
    m = jnp.max(v, axis=1, keepdims=True)
    e = jnp.exp(v - m)
    w = e / jnp.sum(e, axis=1, keepdims=True)

    gcol = jax.lax.broadcasted_iota(jnp.int32, (B_BLK, M_CHUNK), 1) \
        + j * M_CHUNK
    oh = jnp.zeros((B_BLK, M_CHUNK), jnp.float32)
    for k in range(TOPK):
        oh = oh + jnp.where(idx_ref[:, k:k + 1] == gcol, w[:, k:k + 1], 0.0)

    out_ref[:] += jax.lax.dot_general(
        oh, mem_ref[:], (((1,), (0,)), ((), ())),
        preferred_element_type=jnp.float32)


@jax.jit
def kernel(x, memory):
    b, d = x.shape
    m, _ = memory.shape
    m_pad = ((m + M_CHUNK - 1) // M_CHUNK) * M_CHUNK
    mem_p = jnp.pad(memory, ((0, m_pad - m), (0, 0)))
    nb = b // B_BLK
    nj = m_pad // M_CHUNK

    vals, idx = pl.pallas_call(
        functools.partial(_topk_kernel, n_chunks=nj, m_valid=m),
        grid=(nb, nj),
        in_specs=[
            pl.BlockSpec((B_BLK, d), lambda i, j: (i, 0)),
            pl.BlockSpec((M_CHUNK, d), lambda i, j: (j, 0)),
        ],
        out_specs=[
            pl.BlockSpec((B_BLK, KPAD), lambda i, j: (i, 0)),
            pl.BlockSpec((B_BLK, KPAD), lambda i, j: (i, 0)),
        ],
        out_shape=[
            jax.ShapeDtypeStruct((b, KPAD), jnp.float32),
            jax.ShapeDtypeStruct((b, KPAD), jnp.int32),
        ],
        scratch_shapes=[
            pltpu.VMEM((B_BLK, KPAD), jnp.float32),
            pltpu.VMEM((B_BLK, KPAD), jnp.int32),
        ],
        compiler_params=pltpu.CompilerParams(
            dimension_semantics=("arbitrary", "arbitrary")),
    )(x, mem_p)

    recon = pl.pallas_call(
        _recon_kernel,
        grid=(nb, nj),
        in_specs=[
            pl.BlockSpec((B_BLK, KPAD), lambda i, j: (i, 0)),
            pl.BlockSpec((B_BLK, KPAD), lambda i, j: (i, 0)),
            pl.BlockSpec((M_CHUNK, d), lambda i, j: (j, 0)),
        ],
        out_specs=pl.BlockSpec((B_BLK, d), lambda i, j: (i, 0)),
        out_shape=jax.ShapeDtypeStruct((b, d), jnp.float32),
        compiler_params=pltpu.CompilerParams(
            dimension_semantics=("arbitrary", "arbitrary")),
    )(vals, idx, mem_p)

    return recon
